# baseline (speedup 1.0000x reference)
import jax
import jax.numpy as jnp
from jax import lax
from jax.experimental import pallas as pl
from jax.experimental.pallas import tpu as pltpu
from jax.experimental.pallas import tpu_sc as plsc

N = 10000
E = 160000
D = 384
NUM_REL = 2

NPAD = 10240
CHUNK = 128
NCHUNK = D // CHUNK
ACC_ROWS = NPAD + 128
DUMP_ROW = NPAD
NT = 16
EB = 128
NB = 80
EPAD = NT * NB * EB
ROWS_PER_TILE = ACC_ROWS // NT
BN = 256
GRID = NPAD // BN

HALF = NB // 4
CHALF = NB // 2
NSTAGE = NB // HALF
CW = CHUNK


def _sc_aggregate_body(x0, x1, x2, src_hbm, scat_hbm, zeros_hbm,
                       out0, out1, out2,
                       idx_src2, idx_scat2, rows, acc, gsem, ssem):
    cid = lax.axis_index("c")
    sid = lax.axis_index("s")
    xs = [x0, x1, x2]
    outs = [out0, out1, out2]
    for c in range(NCHUNK):
        pltpu.sync_copy(zeros_hbm, acc.at[pl.ds(sid * ROWS_PER_TILE,
                                                ROWS_PER_TILE)])
        plsc.subcore_barrier()
        xc = xs[c]

        HB = EB // 2

        def gissue(j, slot):
            for h in range(2):
                pltpu.async_copy(xc.at[idx_src2.at[2 * j + h]],
                                 rows.at[slot, pl.ds(h * HB, HB), :], gsem)

        def gwait(slot):
            for h in range(2):
                pltpu.make_async_copy(
                    xc.at[idx_src2.at[0]],
                    rows.at[slot, pl.ds(h * HB, HB), :], gsem).wait()

        def sissue(j, slot):
            for h in range(2):
                pltpu.async_copy(rows.at[slot, pl.ds(h * HB, HB), :],
                                 acc.at[idx_scat2.at[2 * j + h]],
                                 ssem, add=True)

        def swait(slot):
            for h in range(2):
                pltpu.make_async_copy(
                    rows.at[slot, pl.ds(h * HB, HB), :],
                    acc.at[idx_scat2.at[0]], ssem).wait()

        for half in range(NSTAGE):
            pltpu.sync_copy(
                src_hbm.at[cid, pl.ds(2 * (sid * NB + half * HALF),
                                      2 * HALF), :],
                idx_src2)
            pltpu.sync_copy(
                scat_hbm.at[cid, pl.ds(2 * (sid * NB + half * HALF),
                                       2 * HALF), :],
                idx_scat2)
            gissue(0, 0)

            @pl.loop(0, HALF // 2)
            def _grp(g):
                j0 = g * 2
                j1 = j0 + 1
                gwait(0)
                sissue(j0, 0)
                gissue(j1, 1)
                swait(0)

                @pl.when(j0 + 2 < HALF)
                def _():
                    gissue(j0 + 2, 0)

                gwait(1)
                sissue(j1, 1)
                swait(1)

        plsc.subcore_barrier()

        @pl.when(sid == 0)
        def _():
            pltpu.sync_copy(acc, outs[c].at[cid])

        plsc.subcore_barrier()


def _sc_counts_body(scat_hbm, ones_hbm, zeros_hbm, out,
                    idx_scat, ones, acc, ssem):
    cid = lax.axis_index("c")
    sid = lax.axis_index("s")
    pltpu.sync_copy(ones_hbm, ones)
    pltpu.sync_copy(zeros_hbm, acc.at[pl.ds(sid * ROWS_PER_TILE,
                                            ROWS_PER_TILE)])
    plsc.subcore_barrier()
    for half in range(NB // CHALF):
        pltpu.sync_copy(
            scat_hbm.at[cid, pl.ds(sid * NB + half * CHALF, CHALF), :],
            idx_scat)

        @pl.loop(0, CHALF // 4)
        def _grp(g):
            hs = []
            for b in range(4):
                j = g * 4 + b
                hs.append(pltpu.async_copy(ones, acc.at[idx_scat.at[j]],
                                           ssem, add=True))
            for h in hs:
                h.wait()

    plsc.subcore_barrier()

    @pl.when(sid == 0)
    def _():
        pltpu.sync_copy(acc, out.at[cid])


def _sc_mesh():
    return plsc.VectorSubcoreMesh(core_axis_name="c", subcore_axis_name="s")


def _sc_aggregate(xch, src_rows, scat_rows, zeros128):
    out_type = tuple(jax.ShapeDtypeStruct((2, ACC_ROWS, CHUNK), jnp.float32)
                     for _ in range(NCHUNK))
    f = pl.kernel(
        _sc_aggregate_body,
        out_type=out_type,
        mesh=_sc_mesh(),
        scratch_types=[
            pltpu.VMEM((2 * HALF, EB // 2), jnp.int32),
            pltpu.VMEM((2 * HALF, EB // 2), jnp.int32),
            pltpu.VMEM((2, EB, CHUNK), jnp.float32),
            pltpu.VMEM_SHARED((ACC_ROWS, CHUNK), jnp.float32),
            pltpu.SemaphoreType.DMA,
            pltpu.SemaphoreType.DMA,
        ],
    )
    return f(xch[0], xch[1], xch[2], src_rows, scat_rows, zeros128)


def _sc_counts(scat_rows, ones128, zeros128):
    f = pl.kernel(
        _sc_counts_body,
        out_type=jax.ShapeDtypeStruct((2, ACC_ROWS, CW), jnp.float32),
        mesh=_sc_mesh(),
        scratch_types=[
            pltpu.VMEM((CHALF, EB), jnp.int32),
            pltpu.VMEM((EB, CW), jnp.float32),
            pltpu.VMEM_SHARED((ACC_ROWS, CW), jnp.float32),
            pltpu.SemaphoreType.DMA,
        ],
    )
    return f(scat_rows, ones128, zeros128)


def _tc_layer_body_factory(relu, split):
    def body(c0, c1, x0, x1, x2,
             p00, p01, p10, p11, p20, p21,
             wcat, bias, *outs):
        xb = jnp.concatenate([x0[...], x1[...], x2[...]], axis=1)

        def mean_for(cref, prs):
            cnt = cref[...][0][:, 0:1]
            inv = 1.0 / jnp.maximum(cnt, 1.0)
            s = jnp.concatenate([p[...][0] for p in prs], axis=1)
            return s * inv

        m0 = mean_for(c0, [p00, p10, p20])
        m1 = mean_for(c1, [p01, p11, p21])
        a = jnp.concatenate([xb, m0, m1], axis=1)
        y = lax.dot_general(a, wcat[...], (((1,), (0,)), ((), ())),
                            preferred_element_type=jnp.float32)
        y = y + bias[...]
        if relu:
            y = jnp.maximum(y, 0.0)
        if split:
            for c in range(NCHUNK):
                outs[c][...] = y[:, c * CHUNK:(c + 1) * CHUNK]
        else:
            outs[0][...] = y

    return body


def _tc_layer(xch, partials, counts, wcat, bias2, relu, split):
    cnt_specs = [pl.BlockSpec((1, BN, CW), lambda n, r=r: (r, n, 0))
                 for r in range(NUM_REL)]
    x_spec = pl.BlockSpec((BN, CHUNK), lambda n: (n, 0))
    w_spec = pl.BlockSpec((3 * D, D), lambda n: (0, 0))
    b_spec = pl.BlockSpec((1, D), lambda n: (0, 0))

    in_specs = list(cnt_specs) + [x_spec] * NCHUNK
    operands = [counts, counts] + list(xch)
    for p in partials:
        in_specs += [pl.BlockSpec((1, BN, CHUNK), lambda n, r=r: (r, n, 0))
                     for r in range(NUM_REL)]
        operands += [p, p]
    in_specs += [w_spec, b_spec]
    operands += [wcat, bias2]

    if split:
        out_shape = tuple(jax.ShapeDtypeStruct((NPAD, CHUNK), jnp.float32)
                          for _ in range(NCHUNK))
        out_specs = tuple(pl.BlockSpec((BN, CHUNK), lambda n: (n, 0))
                          for _ in range(NCHUNK))
    else:
        out_shape = jax.ShapeDtypeStruct((NPAD, D), jnp.float32)
        out_specs = pl.BlockSpec((BN, D), lambda n: (n, 0))

    f = pl.pallas_call(
        _tc_layer_body_factory(relu, split),
        grid=(GRID,),
        in_specs=in_specs,
        out_specs=out_specs,
        out_shape=out_shape,
    )
    return f(*operands)


def kernel(embeddings, edge_index, edge_type, w1, r1, b1, w2, r2, b2, w3, r3, b3):
    src = edge_index[0].astype(jnp.int32)
    dst = edge_index[1].astype(jnp.int32)
    et = edge_type.astype(jnp.int32)
    pad = EPAD - E
    src_list, scat_list = [], []
    for rel in range(NUM_REL):
        mine = et == rel
        g = jnp.where(mine, src, dst)
        s = jnp.where(mine, dst, DUMP_ROW + (dst & 127))
        src_list.append(jnp.concatenate(
            [g, jnp.arange(pad, dtype=jnp.int32) & 8191]
        ).reshape(NT * NB, EB))
        scat_list.append(jnp.concatenate(
            [s, jnp.full((pad,), DUMP_ROW, jnp.int32)]
        ).reshape(NT * NB, EB))
    src_rows = jnp.stack(src_list)
    scat_rows = jnp.stack(scat_list)
    src_rows64 = src_rows.reshape(NUM_REL, NT * NB * 2, EB // 2)
    scat_rows64 = scat_rows.reshape(NUM_REL, NT * NB * 2, EB // 2)
    x = jnp.pad(embeddings, ((0, NPAD - N), (0, 0)))
    xch = [x[:, c * CHUNK:(c + 1) * CHUNK] for c in range(NCHUNK)]
    zeros128 = jnp.zeros((ROWS_PER_TILE, CHUNK), jnp.float32)
    ones128 = jnp.ones((EB, CW), jnp.float32)

    counts = _sc_counts(scat_rows, ones128, zeros128)

    for li, (w, r, b) in enumerate([(w1, r1, b1), (w2, r2, b2), (w3, r3, b3)]):
        wcat = jnp.concatenate([r, w[0], w[1]], axis=0)
        bias2 = b.reshape(1, D)
        partials = _sc_aggregate(xch, src_rows64, scat_rows64, zeros128)
        last = li == 2
        res = _tc_layer(xch, list(partials), counts, wcat, bias2,
                        relu=not last, split=not last)
        if last:
            return res[:N]
        xch = list(res)


# speedup vs baseline: 1.0453x; 1.0453x over previous
import jax
import jax.numpy as jnp
from jax import lax
from jax.experimental import pallas as pl
from jax.experimental.pallas import tpu as pltpu
from jax.experimental.pallas import tpu_sc as plsc

N = 10000
E = 160000
D = 384
NUM_REL = 2

NPAD = 10240
CHUNK = 128
NCHUNK = D // CHUNK
ACC_ROWS = NPAD + 128
DUMP_ROW = NPAD
NT = 16
EB = 128
NB = 80
EPAD = NT * NB * EB
ROWS_PER_TILE = ACC_ROWS // NT
BN = 256
GRID = NPAD // BN

HALF = NB // 2
NSTAGE = NB // HALF
CW = CHUNK


def _sc_aggregate_body(x0, x1, x2, src_hbm, scat_hbm, zeros_hbm,
                       out0, out1, out2,
                       idx_src, idx_scat, rows, acc, gsem, ssem):
    cid = lax.axis_index("c")
    sid = lax.axis_index("s")
    xs = [x0, x1, x2]
    outs = [out0, out1, out2]
    for c in range(NCHUNK):
        pltpu.sync_copy(zeros_hbm, acc.at[pl.ds(sid * ROWS_PER_TILE,
                                                ROWS_PER_TILE)])
        plsc.subcore_barrier()
        xc = xs[c]

        def gissue(j, slot):
            pltpu.async_copy(xc.at[idx_src.at[j]], rows.at[slot], gsem)

        def gwait(slot):
            pltpu.make_async_copy(xc.at[idx_src.at[0]],
                                  rows.at[slot], gsem).wait()

        def sissue(j, slot):
            pltpu.async_copy(rows.at[slot], acc.at[idx_scat.at[j]],
                             ssem, add=True)

        def swait(slot):
            pltpu.make_async_copy(rows.at[slot],
                                  acc.at[idx_scat.at[0]], ssem).wait()

        for half in range(NSTAGE):
            pltpu.sync_copy(
                src_hbm.at[cid, pl.ds(sid * NB + half * HALF, HALF), :],
                idx_src)
            pltpu.sync_copy(
                scat_hbm.at[cid, pl.ds(sid * NB + half * HALF, HALF), :],
                idx_scat)
            gissue(0, 0)

            @pl.loop(0, HALF // 2)
            def _grp(g):
                j0 = g * 2
                j1 = j0 + 1
                gwait(0)
                sissue(j0, 0)
                gissue(j1, 1)
                swait(0)

                @pl.when(j0 + 2 < HALF)
                def _():
                    gissue(j0 + 2, 0)

                gwait(1)
                sissue(j1, 1)
                swait(1)

        plsc.subcore_barrier()

        @pl.when(sid == 0)
        def _():
            pltpu.sync_copy(acc, outs[c].at[cid])

        plsc.subcore_barrier()


def _sc_counts_body(scat_hbm, ones_hbm, zeros_hbm, out,
                    idx_scat, ones, acc, ssem):
    cid = lax.axis_index("c")
    sid = lax.axis_index("s")
    pltpu.sync_copy(ones_hbm, ones)
    pltpu.sync_copy(zeros_hbm, acc.at[pl.ds(sid * ROWS_PER_TILE,
                                            ROWS_PER_TILE)])
    plsc.subcore_barrier()
    for half in range(NSTAGE):
        pltpu.sync_copy(
            scat_hbm.at[cid, pl.ds(sid * NB + half * HALF, HALF), :],
            idx_scat)

        @pl.loop(0, HALF // 4)
        def _grp(g):
            hs = []
            for b in range(4):
                j = g * 4 + b
                hs.append(pltpu.async_copy(ones, acc.at[idx_scat.at[j]],
                                           ssem, add=True))
            for h in hs:
                h.wait()

    plsc.subcore_barrier()

    @pl.when(sid == 0)
    def _():
        pltpu.sync_copy(acc, out.at[cid])


def _sc_mesh():
    return plsc.VectorSubcoreMesh(core_axis_name="c", subcore_axis_name="s")


def _sc_aggregate(xch, src_rows, scat_rows, zeros128):
    out_type = tuple(jax.ShapeDtypeStruct((2, ACC_ROWS, CHUNK), jnp.float32)
                     for _ in range(NCHUNK))
    f = pl.kernel(
        _sc_aggregate_body,
        out_type=out_type,
        mesh=_sc_mesh(),
        scratch_types=[
            pltpu.VMEM((HALF, EB), jnp.int32),
            pltpu.VMEM((HALF, EB), jnp.int32),
            pltpu.VMEM((2, EB, CHUNK), jnp.float32),
            pltpu.VMEM_SHARED((ACC_ROWS, CHUNK), jnp.float32),
            pltpu.SemaphoreType.DMA,
            pltpu.SemaphoreType.DMA,
        ],
    )
    return f(xch[0], xch[1], xch[2], src_rows, scat_rows, zeros128)


def _sc_counts(scat_rows, ones128, zeros128):
    f = pl.kernel(
        _sc_counts_body,
        out_type=jax.ShapeDtypeStruct((2, ACC_ROWS, CW), jnp.float32),
        mesh=_sc_mesh(),
        scratch_types=[
            pltpu.VMEM((HALF, EB), jnp.int32),
            pltpu.VMEM((EB, CW), jnp.float32),
            pltpu.VMEM_SHARED((ACC_ROWS, CW), jnp.float32),
            pltpu.SemaphoreType.DMA,
        ],
    )
    return f(scat_rows, ones128, zeros128)


def _tc_layer_body_factory(relu, split):
    def body(c0, c1, x0, x1, x2,
             p00, p01, p10, p11, p20, p21,
             wcat, bias, *outs):
        xb = jnp.concatenate([x0[...], x1[...], x2[...]], axis=1)

        def mean_for(cref, prs):
            cnt = cref[...][0][:, 0:1]
            inv = 1.0 / jnp.maximum(cnt, 1.0)
            s = jnp.concatenate([p[...][0] for p in prs], axis=1)
            return s * inv

        m0 = mean_for(c0, [p00, p10, p20])
        m1 = mean_for(c1, [p01, p11, p21])
        a = jnp.concatenate([xb, m0, m1], axis=1)
        y = lax.dot_general(a, wcat[...], (((1,), (0,)), ((), ())),
                            preferred_element_type=jnp.float32)
        y = y + bias[...]
        if relu:
            y = jnp.maximum(y, 0.0)
        if split:
            for c in range(NCHUNK):
                outs[c][...] = y[:, c * CHUNK:(c + 1) * CHUNK]
        else:
            outs[0][...] = y

    return body


def _tc_layer(xch, partials, counts, wcat, bias2, relu, split):
    cnt_specs = [pl.BlockSpec((1, BN, CW), lambda n, r=r: (r, n, 0))
                 for r in range(NUM_REL)]
    x_spec = pl.BlockSpec((BN, CHUNK), lambda n: (n, 0))
    w_spec = pl.BlockSpec((3 * D, D), lambda n: (0, 0))
    b_spec = pl.BlockSpec((1, D), lambda n: (0, 0))

    in_specs = list(cnt_specs) + [x_spec] * NCHUNK
    operands = [counts, counts] + list(xch)
    for p in partials:
        in_specs += [pl.BlockSpec((1, BN, CHUNK), lambda n, r=r: (r, n, 0))
                     for r in range(NUM_REL)]
        operands += [p, p]
    in_specs += [w_spec, b_spec]
    operands += [wcat, bias2]

    if split:
        out_shape = tuple(jax.ShapeDtypeStruct((NPAD, CHUNK), jnp.float32)
                          for _ in range(NCHUNK))
        out_specs = tuple(pl.BlockSpec((BN, CHUNK), lambda n: (n, 0))
                          for _ in range(NCHUNK))
    else:
        out_shape = jax.ShapeDtypeStruct((NPAD, D), jnp.float32)
        out_specs = pl.BlockSpec((BN, D), lambda n: (n, 0))

    f = pl.pallas_call(
        _tc_layer_body_factory(relu, split),
        grid=(GRID,),
        in_specs=in_specs,
        out_specs=out_specs,
        out_shape=out_shape,
    )
    return f(*operands)


def kernel(embeddings, edge_index, edge_type, w1, r1, b1, w2, r2, b2, w3, r3, b3):
    src = edge_index[0].astype(jnp.int32)
    dst = edge_index[1].astype(jnp.int32)
    et = edge_type.astype(jnp.int32)
    pad = EPAD - E
    src_list, scat_list = [], []
    for rel in range(NUM_REL):
        mine = et == rel
        g = jnp.where(mine, src, dst)
        s = jnp.where(mine, dst, DUMP_ROW + (dst & 127))
        src_list.append(jnp.concatenate(
            [g, jnp.arange(pad, dtype=jnp.int32) & 8191]
        ).reshape(NT * NB, EB))
        scat_list.append(jnp.concatenate(
            [s, jnp.full((pad,), DUMP_ROW, jnp.int32)]
        ).reshape(NT * NB, EB))
    src_rows = jnp.stack(src_list)
    scat_rows = jnp.stack(scat_list)
    x = jnp.pad(embeddings, ((0, NPAD - N), (0, 0)))
    xch = [x[:, c * CHUNK:(c + 1) * CHUNK] for c in range(NCHUNK)]
    zeros128 = jnp.zeros((ROWS_PER_TILE, CHUNK), jnp.float32)
    ones128 = jnp.ones((EB, CW), jnp.float32)

    counts = _sc_counts(scat_rows, ones128, zeros128)

    for li, (w, r, b) in enumerate([(w1, r1, b1), (w2, r2, b2), (w3, r3, b3)]):
        wcat = jnp.concatenate([r, w[0], w[1]], axis=0)
        bias2 = b.reshape(1, D)
        partials = _sc_aggregate(xch, src_rows, scat_rows, zeros128)
        last = li == 2
        res = _tc_layer(xch, list(partials), counts, wcat, bias2,
                        relu=not last, split=not last)
        if last:
            return res[:N]
        xch = list(res)


# speedup vs baseline: 1.0466x; 1.0012x over previous
import jax
import jax.numpy as jnp
from jax import lax
from jax.experimental import pallas as pl
from jax.experimental.pallas import tpu as pltpu
from jax.experimental.pallas import tpu_sc as plsc

N = 10000
E = 160000
D = 384
NUM_REL = 2

NPAD = 10240
CHUNK = 128
NCHUNK = D // CHUNK
ACC_ROWS = NPAD + 128
DUMP_ROW = NPAD
NT = 16
EB = 128
NB = 80
EPAD = NT * NB * EB
ROWS_PER_TILE = ACC_ROWS // NT
BN = 256
GRID = NPAD // BN

HALF = NB // 2
NSTAGE = NB // HALF
CW = CHUNK


def _sc_aggregate_body(x0, x1, x2, src_hbm, scat_hbm, zeros_hbm,
                       out0, out1, out2,
                       idx_src, idx_scat, rows, acc, gsem, ssem):
    cid = lax.axis_index("c")
    sid = lax.axis_index("s")
    xs = [x0, x1, x2]
    outs = [out0, out1, out2]
    for c in range(NCHUNK):
        pltpu.sync_copy(zeros_hbm, acc.at[pl.ds(sid * ROWS_PER_TILE,
                                                ROWS_PER_TILE)])
        plsc.subcore_barrier()
        xc = xs[c]

        def gissue(j, slot):
            pltpu.async_copy(xc.at[idx_src.at[j]], rows.at[slot], gsem)

        def gwait(slot):
            pltpu.make_async_copy(xc.at[idx_src.at[0]],
                                  rows.at[slot], gsem).wait()

        def sissue(j, slot):
            pltpu.async_copy(rows.at[slot], acc.at[idx_scat.at[j]],
                             ssem, add=True)

        def swait(slot):
            pltpu.make_async_copy(rows.at[slot],
                                  acc.at[idx_scat.at[0]], ssem).wait()

        for half in range(NSTAGE):
            pltpu.sync_copy(
                src_hbm.at[cid, pl.ds(sid * NB + half * HALF, HALF), :],
                idx_src)
            pltpu.sync_copy(
                scat_hbm.at[cid, pl.ds(sid * NB + half * HALF, HALF), :],
                idx_scat)
            gissue(0, 0)

            @pl.loop(0, HALF // 2)
            def _grp(g):
                j0 = g * 2
                j1 = j0 + 1
                gwait(0)
                sissue(j0, 0)
                gissue(j1, 1)
                swait(0)

                @pl.when(j0 + 2 < HALF)
                def _():
                    gissue(j0 + 2, 0)

                gwait(1)
                sissue(j1, 1)
                swait(1)

        plsc.subcore_barrier()

        @pl.when(sid == 0)
        def _():
            pltpu.sync_copy(acc, outs[c].at[cid])

        plsc.subcore_barrier()


def _sc_counts_body(scat_hbm, ones_hbm, zeros_hbm, out,
                    idx_scat, ones, acc, ssem):
    cid = lax.axis_index("c")
    sid = lax.axis_index("s")
    pltpu.sync_copy(ones_hbm, ones)
    pltpu.sync_copy(zeros_hbm, acc.at[pl.ds(sid * ROWS_PER_TILE,
                                            ROWS_PER_TILE)])
    plsc.subcore_barrier()
    for half in range(NSTAGE):
        pltpu.sync_copy(
            scat_hbm.at[cid, pl.ds(sid * NB + half * HALF, HALF), :],
            idx_scat)

        @pl.loop(0, HALF // 4)
        def _grp(g):
            hs = []
            for b in range(4):
                j = g * 4 + b
                hs.append(pltpu.async_copy(ones, acc.at[idx_scat.at[j]],
                                           ssem, add=True))
            for h in hs:
                h.wait()

    plsc.subcore_barrier()

    @pl.when(sid == 0)
    def _():
        pltpu.sync_copy(acc, out.at[cid])


def _sc_mesh():
    return plsc.VectorSubcoreMesh(core_axis_name="c", subcore_axis_name="s")


def _sc_aggregate(xch, src_rows, scat_rows, zeros128):
    out_type = tuple(jax.ShapeDtypeStruct((2, ACC_ROWS, CHUNK), jnp.float32)
                     for _ in range(NCHUNK))
    f = pl.kernel(
        _sc_aggregate_body,
        out_type=out_type,
        mesh=_sc_mesh(),
        scratch_types=[
            pltpu.VMEM((HALF, EB), jnp.int32),
            pltpu.VMEM((HALF, EB), jnp.int32),
            pltpu.VMEM((2, EB, CHUNK), jnp.float32),
            pltpu.VMEM_SHARED((ACC_ROWS, CHUNK), jnp.float32),
            pltpu.SemaphoreType.DMA,
            pltpu.SemaphoreType.DMA,
        ],
    )
    return f(xch[0], xch[1], xch[2], src_rows, scat_rows, zeros128)


def _sc_counts(scat_rows, ones128, zeros128):
    f = pl.kernel(
        _sc_counts_body,
        out_type=jax.ShapeDtypeStruct((2, ACC_ROWS, CW), jnp.float32),
        mesh=_sc_mesh(),
        scratch_types=[
            pltpu.VMEM((HALF, EB), jnp.int32),
            pltpu.VMEM((EB, CW), jnp.float32),
            pltpu.VMEM_SHARED((ACC_ROWS, CW), jnp.float32),
            pltpu.SemaphoreType.DMA,
        ],
    )
    return f(scat_rows, ones128, zeros128)


def _tc_layer_body_factory(relu, split):
    def body(c0, c1, x0, x1, x2,
             p00, p01, p10, p11, p20, p21,
             wcat, bias, *outs):
        xb = jnp.concatenate([x0[...], x1[...], x2[...]], axis=1)

        def mean_for(cref, prs):
            cnt = cref[...][0][:, 0:1]
            inv = 1.0 / jnp.maximum(cnt, 1.0)
            s = jnp.concatenate([p[...][0] for p in prs], axis=1)
            return s * inv

        m0 = mean_for(c0, [p00, p10, p20])
        m1 = mean_for(c1, [p01, p11, p21])
        a = jnp.concatenate([xb, m0, m1], axis=1)
        y = lax.dot_general(a, wcat[...], (((1,), (0,)), ((), ())),
                            preferred_element_type=jnp.float32)
        y = y + bias[...]
        if relu:
            y = jnp.maximum(y, 0.0)
        if split:
            for c in range(NCHUNK):
                outs[c][...] = y[:, c * CHUNK:(c + 1) * CHUNK]
        else:
            outs[0][...] = y

    return body


def _tc_layer(xch, partials, counts, wcat, bias2, relu, split):
    cnt_specs = [pl.BlockSpec((1, BN, CW), lambda n, r=r: (r, n, 0))
                 for r in range(NUM_REL)]
    x_spec = pl.BlockSpec((BN, CHUNK), lambda n: (n, 0))
    w_spec = pl.BlockSpec((3 * D, D), lambda n: (0, 0))
    b_spec = pl.BlockSpec((1, D), lambda n: (0, 0))

    in_specs = list(cnt_specs) + [x_spec] * NCHUNK
    operands = [counts, counts] + list(xch)
    for p in partials:
        in_specs += [pl.BlockSpec((1, BN, CHUNK), lambda n, r=r: (r, n, 0))
                     for r in range(NUM_REL)]
        operands += [p, p]
    in_specs += [w_spec, b_spec]
    operands += [wcat, bias2]

    if split:
        out_shape = tuple(jax.ShapeDtypeStruct((NPAD, CHUNK), jnp.float32)
                          for _ in range(NCHUNK))
        out_specs = tuple(pl.BlockSpec((BN, CHUNK), lambda n: (n, 0))
                          for _ in range(NCHUNK))
    else:
        out_shape = jax.ShapeDtypeStruct((NPAD, D), jnp.float32)
        out_specs = pl.BlockSpec((BN, D), lambda n: (n, 0))

    f = pl.pallas_call(
        _tc_layer_body_factory(relu, split),
        grid=(GRID,),
        in_specs=in_specs,
        out_specs=out_specs,
        out_shape=out_shape,
    )
    return f(*operands)


def kernel(embeddings, edge_index, edge_type, w1, r1, b1, w2, r2, b2, w3, r3, b3):
    src = edge_index[0].astype(jnp.int32)
    dst = edge_index[1].astype(jnp.int32)
    et = edge_type.astype(jnp.int32)
    pad = EPAD - E
    src_list, scat_list = [], []
    for rel in range(NUM_REL):
        mine = et == rel
        g = jnp.where(mine, src, dst)
        s = jnp.where(mine, dst, DUMP_ROW + (dst & 127))
        src_list.append(jnp.concatenate(
            [g, jnp.arange(pad, dtype=jnp.int32) & 8191]
        ).reshape(NT * NB, EB))
        scat_list.append(jnp.concatenate(
            [s, DUMP_ROW + (jnp.arange(pad, dtype=jnp.int32) & 127)]
        ).reshape(NT * NB, EB))
    src_rows = jnp.stack(src_list)
    scat_rows = jnp.stack(scat_list)
    x = jnp.pad(embeddings, ((0, NPAD - N), (0, 0)))
    xch = [x[:, c * CHUNK:(c + 1) * CHUNK] for c in range(NCHUNK)]
    zeros128 = jnp.zeros((ROWS_PER_TILE, CHUNK), jnp.float32)
    ones128 = jnp.ones((EB, CW), jnp.float32)

    counts = _sc_counts(scat_rows, ones128, zeros128)

    for li, (w, r, b) in enumerate([(w1, r1, b1), (w2, r2, b2), (w3, r3, b3)]):
        wcat = jnp.concatenate([r, w[0], w[1]], axis=0)
        bias2 = b.reshape(1, D)
        partials = _sc_aggregate(xch, src_rows, scat_rows, zeros128)
        last = li == 2
        res = _tc_layer(xch, list(partials), counts, wcat, bias2,
                        relu=not last, split=not last)
        if last:
            return res[:N]
        xch = list(res)
